# Initial kernel scaffold; baseline (speedup 1.0000x reference)
import jax
import jax.numpy as jnp
from jax.experimental import pallas as pl


def kernel(edge_index, x, W_cheb, b_cheb, gamma, beta, W_match, b_match):
    raise NotImplementedError("write your pallas kernel here")


# speedup vs baseline: 4.8846x; 4.8846x over previous
import functools

import jax
import jax.numpy as jnp
from jax import lax
from jax.experimental import pallas as pl
from jax.experimental.pallas import tpu as pltpu
from jax.experimental.pallas import tpu_sc as plsc

_N = 10000
_E = 320000
_C = 128
_EPS = 1e-5
_NP = 10240
_B = 128
_NCHUNK = 2560
_EPAD = _NCHUNK * _B
_NC = 2
_NS = 16
_GT = _NCHUNK // (_NC * _NS)
_ZR = _NP // _NS
_DW = 16

def _prop_body(hs, col2d, row2d, out, acc, colv, rowv, gbuf, gsem):
    c = lax.axis_index("c")
    s = lax.axis_index("s")

    def fill(i, carry):
        for j in range(_C // 16):
            gbuf[i, pl.ds(j * 16, 16)] = jnp.zeros((16,), jnp.float32)
        return carry

    lax.fori_loop(0, _B, fill, 0)

    def zero(k, carry):
        pltpu.sync_copy(gbuf, acc.at[pl.ds(s * _ZR + k * _B, _B)])
        return carry

    lax.fori_loop(0, _ZR // _B, zero, 0)

    base = (c * _NS + s) * _GT
    pltpu.sync_copy(col2d.at[pl.ds(base, _GT)], colv)
    pltpu.sync_copy(row2d.at[pl.ds(base, _GT)], rowv)
    plsc.subcore_barrier()

    def chunk(g, carry):
        pltpu.async_copy(hs.at[colv.at[g]], gbuf, gsem).wait()
        pltpu.sync_copy(gbuf, acc.at[rowv.at[g]], add=True)
        return carry

    lax.fori_loop(0, _GT, chunk, 0)
    plsc.subcore_barrier()
    pltpu.sync_copy(acc.at[pl.ds(s * _ZR, _ZR)],
                    out.at[pl.ds(c * _NP + s * _ZR, _ZR)])


@functools.cache
def _prop_kernel():
    mesh = plsc.VectorSubcoreMesh(core_axis_name="c", subcore_axis_name="s")
    return pl.kernel(
        _prop_body,
        out_type=jax.ShapeDtypeStruct((_NC * _NP, _C), jnp.float32),
        mesh=mesh,
        scratch_types=[
            pltpu.VMEM_SHARED((_NP, _C), jnp.float32),
            pltpu.VMEM((_GT, _B), jnp.int32),
            pltpu.VMEM((_GT, _B), jnp.int32),
            pltpu.VMEM((_B, _C), jnp.float32),
            pltpu.SemaphoreType.DMA,
        ],
    )


def _tc1_body(degp, x, dinv_o, hs1_o):
    d = degp[0, :, 0:1] + degp[1, :, 0:1]
    dinv = jnp.where(d > 0.0, lax.rsqrt(d), 0.0)
    dinv_o[...] = dinv
    hs1_o[...] = x[...] * dinv


def _tc1(degp, x_pad):
    return pl.pallas_call(
        _tc1_body,
        out_shape=[
            jax.ShapeDtypeStruct((_NP, 1), jnp.float32),
            jax.ShapeDtypeStruct((_NP, _C), jnp.float32),
        ],
    )(degp, x_pad)


def _tc2_body(S, dinv, x, Wm, bm, tx1_o, hs2_o, id_o):
    ssum = S[0] + S[1]
    d = dinv[...]
    tx1 = -(d * ssum)
    tx1_o[...] = tx1
    hs2_o[...] = d * tx1
    id_o[...] = jnp.dot(x[...], Wm[...],
                        preferred_element_type=jnp.float32) + bm[...]


def _tc2(S1, dinv, x_pad, W_match, b_match):
    return pl.pallas_call(
        _tc2_body,
        out_shape=[
            jax.ShapeDtypeStruct((_NP, _C), jnp.float32),
            jax.ShapeDtypeStruct((_NP, _C), jnp.float32),
            jax.ShapeDtypeStruct((_NP, _C), jnp.float32),
        ],
    )(S1, dinv, x_pad, W_match, b_match)


_RB = 1024
_NBLK = _NP // _RB


def _tc3_body(S, dinv, x, tx1, Wc, bc, raw_o, ps_o, pq_o):
    i = pl.program_id(0)
    ssum = S[0] + S[1]
    d = dinv[...]
    xv = x[...]
    tx1v = tx1[...]
    tx2 = -2.0 * (d * ssum) - xv
    raw = (jnp.dot(xv, Wc[0], preferred_element_type=jnp.float32)
           + jnp.dot(tx1v, Wc[1], preferred_element_type=jnp.float32)
           + jnp.dot(tx2, Wc[2], preferred_element_type=jnp.float32)
           + bc[...])
    rows = lax.broadcasted_iota(jnp.int32, (_RB, _C), 0) + i * _RB
    raw = jnp.where(rows < _N, raw, 0.0)
    raw_o[...] = raw
    ps_o[...] = jnp.sum(raw, axis=0, keepdims=True)[None]
    pq_o[...] = jnp.sum(raw * raw, axis=0, keepdims=True)[None]


def _tc3(S2, dinv, x_pad, tx1, W_cheb, b_cheb):
    return pl.pallas_call(
        _tc3_body,
        grid=(_NBLK,),
        in_specs=[
            pl.BlockSpec((2, _RB, _C), lambda i: (0, i, 0)),
            pl.BlockSpec((_RB, 1), lambda i: (i, 0)),
            pl.BlockSpec((_RB, _C), lambda i: (i, 0)),
            pl.BlockSpec((_RB, _C), lambda i: (i, 0)),
            pl.BlockSpec((3, _C, _C), lambda i: (0, 0, 0)),
            pl.BlockSpec((1, _C), lambda i: (0, 0)),
        ],
        out_specs=[
            pl.BlockSpec((_RB, _C), lambda i: (i, 0)),
            pl.BlockSpec((1, 1, _C), lambda i: (i, 0, 0)),
            pl.BlockSpec((1, 1, _C), lambda i: (i, 0, 0)),
        ],
        out_shape=[
            jax.ShapeDtypeStruct((_NP, _C), jnp.float32),
            jax.ShapeDtypeStruct((_NBLK, 1, _C), jnp.float32),
            jax.ShapeDtypeStruct((_NBLK, 1, _C), jnp.float32),
        ],
    )(S2, dinv, x_pad, tx1, W_cheb, b_cheb)


def _tc4_body(raw, ident, ps, pq, gamma, beta, out_o):
    mean = jnp.sum(ps[...], axis=0) * (1.0 / _N)
    var = jnp.sum(pq[...], axis=0) * (1.0 / _N) - mean * mean
    inv = lax.rsqrt(var + _EPS)
    y = (raw[...] - mean) * inv * gamma[...] + beta[...]
    out_o[...] = jnp.maximum(y, 0.0) + ident[...]


def _tc4(raw, ident, ps, pq, gamma, beta):
    return pl.pallas_call(
        _tc4_body,
        grid=(_NBLK,),
        in_specs=[
            pl.BlockSpec((_RB, _C), lambda i: (i, 0)),
            pl.BlockSpec((_RB, _C), lambda i: (i, 0)),
            pl.BlockSpec((_NBLK, 1, _C), lambda i: (0, 0, 0)),
            pl.BlockSpec((_NBLK, 1, _C), lambda i: (0, 0, 0)),
            pl.BlockSpec((1, _C), lambda i: (0, 0)),
            pl.BlockSpec((1, _C), lambda i: (0, 0)),
        ],
        out_specs=pl.BlockSpec((_RB, _C), lambda i: (i, 0)),
        out_shape=jax.ShapeDtypeStruct((_NP, _C), jnp.float32),
    )(raw, ident, ps, pq, gamma, beta)


def kernel(edge_index, x, W_cheb, b_cheb, gamma, beta, W_match, b_match):
    ei = edge_index.astype(jnp.int32)
    pad = jnp.full((_EPAD - _E,), _N, jnp.int32)
    row2d = jnp.concatenate([ei[0], pad]).reshape(_NCHUNK, _B)
    col2d = jnp.concatenate([ei[1], pad]).reshape(_NCHUNK, _B)
    x_pad = jnp.pad(x, ((0, _NP - _N), (0, 0)))

    ones = jnp.ones((_NP, _C), jnp.float32)
    degp = _prop_kernel()(ones, col2d, row2d).reshape(_NC, _NP, _C)
    dinv, hs1 = _tc1(degp, x_pad)
    S1 = _prop_kernel()(hs1, col2d, row2d).reshape(_NC, _NP, _C)
    tx1, hs2, ident = _tc2(S1, dinv, x_pad, W_match,
                           b_match.reshape(1, _C))
    S2 = _prop_kernel()(hs2, col2d, row2d).reshape(_NC, _NP, _C)
    raw, ps, pq = _tc3(S2, dinv, x_pad, tx1, W_cheb,
                       b_cheb.reshape(1, _C))
    out = _tc4(raw, ident, ps, pq, gamma.reshape(1, _C),
               beta.reshape(1, _C))
    return out[:_N]


# speedup vs baseline: 12.1144x; 2.4801x over previous
import functools

import jax
import jax.numpy as jnp
from jax import lax
from jax.experimental import pallas as pl
from jax.experimental.pallas import tpu as pltpu
from jax.experimental.pallas import tpu_sc as plsc

_N = 10000
_E = 320000
_C = 128
_EPS = 1e-5
_NP = 10240
_B = 128
_NCHUNK = 2560
_EPAD = _NCHUNK * _B
_NC = 2
_NS = 16
_GT = _NCHUNK // (_NC * _NS)
_ZR = _NP // _NS
_DW = 16

def _prop_body(hs, col2d, row2d, out, acc, colv, rowv, gbuf, gsem):
    c = lax.axis_index("c")
    s = lax.axis_index("s")

    def fill(i, carry):
        for j in range(_C // 16):
            gbuf[i, pl.ds(j * 16, 16)] = jnp.zeros((16,), jnp.float32)
        return carry

    lax.fori_loop(0, _B, fill, 0)

    def zero(k, carry):
        pltpu.sync_copy(gbuf, acc.at[pl.ds(s * _ZR + k * _B, _B)])
        return carry

    lax.fori_loop(0, _ZR // _B, zero, 0)

    base = (c * _NS + s) * _GT
    pltpu.sync_copy(col2d.at[pl.ds(base, _GT)], colv)
    pltpu.sync_copy(row2d.at[pl.ds(base, _GT)], rowv)
    plsc.subcore_barrier()

    def chunk(g, carry):
        pltpu.async_copy(hs.at[colv.at[g]], gbuf, gsem).wait()
        pltpu.sync_copy(gbuf, acc.at[rowv.at[g]], add=True)
        return carry

    lax.fori_loop(0, _GT, chunk, 0)
    plsc.subcore_barrier()
    pltpu.sync_copy(acc.at[pl.ds(s * _ZR, _ZR)],
                    out.at[pl.ds(c * _NP + s * _ZR, _ZR)])


@functools.cache
def _prop_kernel():
    mesh = plsc.VectorSubcoreMesh(core_axis_name="c", subcore_axis_name="s")
    return pl.kernel(
        _prop_body,
        out_type=jax.ShapeDtypeStruct((_NC * _NP, _C), jnp.float32),
        mesh=mesh,
        scratch_types=[
            pltpu.VMEM_SHARED((_NP, _C), jnp.float32),
            pltpu.VMEM((_GT, _B), jnp.int32),
            pltpu.VMEM((_GT, _B), jnp.int32),
            pltpu.VMEM((_B, _C), jnp.float32),
            pltpu.SemaphoreType.DMA,
        ],
    )


def _tc1_body(degp, x, dinv_o, hs1_o):
    d = degp[0, :, 0:1] + degp[1, :, 0:1]
    dinv = jnp.where(d > 0.0, lax.rsqrt(d), 0.0)
    dinv_o[...] = dinv
    hs1_o[...] = x[...] * dinv


def _tc1(degp, x_pad):
    return pl.pallas_call(
        _tc1_body,
        out_shape=[
            jax.ShapeDtypeStruct((_NP, 1), jnp.float32),
            jax.ShapeDtypeStruct((_NP, _C), jnp.float32),
        ],
    )(degp, x_pad)


def _tc2_body(S, dinv, x, Wm, bm, tx1_o, hs2_o, id_o):
    ssum = S[0] + S[1]
    d = dinv[...]
    tx1 = -(d * ssum)
    tx1_o[...] = tx1
    hs2_o[...] = d * tx1
    id_o[...] = jnp.dot(x[...], Wm[...],
                        preferred_element_type=jnp.float32) + bm[...]


def _tc2(S1, dinv, x_pad, W_match, b_match):
    return pl.pallas_call(
        _tc2_body,
        out_shape=[
            jax.ShapeDtypeStruct((_NP, _C), jnp.float32),
            jax.ShapeDtypeStruct((_NP, _C), jnp.float32),
            jax.ShapeDtypeStruct((_NP, _C), jnp.float32),
        ],
    )(S1, dinv, x_pad, W_match, b_match)


_RB = 1024
_NBLK = _NP // _RB


def _tc3_body(S, dinv, x, tx1, Wc, bc, raw_o, ps_o, pq_o):
    i = pl.program_id(0)
    ssum = S[0] + S[1]
    d = dinv[...]
    xv = x[...]
    tx1v = tx1[...]
    tx2 = -2.0 * (d * ssum) - xv
    raw = (jnp.dot(xv, Wc[0], preferred_element_type=jnp.float32)
           + jnp.dot(tx1v, Wc[1], preferred_element_type=jnp.float32)
           + jnp.dot(tx2, Wc[2], preferred_element_type=jnp.float32)
           + bc[...])
    rows = lax.broadcasted_iota(jnp.int32, (_RB, _C), 0) + i * _RB
    raw = jnp.where(rows < _N, raw, 0.0)
    raw_o[...] = raw
    ps_o[...] = jnp.sum(raw, axis=0, keepdims=True)[None]
    pq_o[...] = jnp.sum(raw * raw, axis=0, keepdims=True)[None]


def _tc3(S2, dinv, x_pad, tx1, W_cheb, b_cheb):
    return pl.pallas_call(
        _tc3_body,
        grid=(_NBLK,),
        in_specs=[
            pl.BlockSpec((2, _RB, _C), lambda i: (0, i, 0)),
            pl.BlockSpec((_RB, 1), lambda i: (i, 0)),
            pl.BlockSpec((_RB, _C), lambda i: (i, 0)),
            pl.BlockSpec((_RB, _C), lambda i: (i, 0)),
            pl.BlockSpec((3, _C, _C), lambda i: (0, 0, 0)),
            pl.BlockSpec((1, _C), lambda i: (0, 0)),
        ],
        out_specs=[
            pl.BlockSpec((_RB, _C), lambda i: (i, 0)),
            pl.BlockSpec((1, 1, _C), lambda i: (i, 0, 0)),
            pl.BlockSpec((1, 1, _C), lambda i: (i, 0, 0)),
        ],
        out_shape=[
            jax.ShapeDtypeStruct((_NP, _C), jnp.float32),
            jax.ShapeDtypeStruct((_NBLK, 1, _C), jnp.float32),
            jax.ShapeDtypeStruct((_NBLK, 1, _C), jnp.float32),
        ],
    )(S2, dinv, x_pad, tx1, W_cheb, b_cheb)


def _tc4_body(raw, ident, ps, pq, gamma, beta, out_o):
    mean = jnp.sum(ps[...], axis=0) * (1.0 / _N)
    var = jnp.sum(pq[...], axis=0) * (1.0 / _N) - mean * mean
    inv = lax.rsqrt(var + _EPS)
    y = (raw[...] - mean) * inv * gamma[...] + beta[...]
    out_o[...] = jnp.maximum(y, 0.0) + ident[...]


def _tc4(raw, ident, ps, pq, gamma, beta):
    return pl.pallas_call(
        _tc4_body,
        grid=(_NBLK,),
        in_specs=[
            pl.BlockSpec((_RB, _C), lambda i: (i, 0)),
            pl.BlockSpec((_RB, _C), lambda i: (i, 0)),
            pl.BlockSpec((_NBLK, 1, _C), lambda i: (0, 0, 0)),
            pl.BlockSpec((_NBLK, 1, _C), lambda i: (0, 0, 0)),
            pl.BlockSpec((1, _C), lambda i: (0, 0)),
            pl.BlockSpec((1, _C), lambda i: (0, 0)),
        ],
        out_specs=pl.BlockSpec((_RB, _C), lambda i: (i, 0)),
        out_shape=jax.ShapeDtypeStruct((_NP, _C), jnp.float32),
    )(raw, ident, ps, pq, gamma, beta)


def kernel(edge_index, x, W_cheb, b_cheb, gamma, beta, W_match, b_match):
    ei = edge_index.astype(jnp.int32)
    pad = _N + (jnp.arange(_EPAD - _E, dtype=jnp.int32) % (_NP - _N))
    row2d = jnp.concatenate([ei[0], pad]).reshape(_NCHUNK, _B)
    col2d = jnp.concatenate([ei[1], pad]).reshape(_NCHUNK, _B)
    x_pad = jnp.pad(x, ((0, _NP - _N), (0, 0)))

    ones = jnp.ones((_NP, _C), jnp.float32)
    degp = _prop_kernel()(ones, col2d, row2d).reshape(_NC, _NP, _C)
    dinv, hs1 = _tc1(degp, x_pad)
    S1 = _prop_kernel()(hs1, col2d, row2d).reshape(_NC, _NP, _C)
    tx1, hs2, ident = _tc2(S1, dinv, x_pad, W_match,
                           b_match.reshape(1, _C))
    S2 = _prop_kernel()(hs2, col2d, row2d).reshape(_NC, _NP, _C)
    raw, ps, pq = _tc3(S2, dinv, x_pad, tx1, W_cheb,
                       b_cheb.reshape(1, _C))
    out = _tc4(raw, ident, ps, pq, gamma.reshape(1, _C),
               beta.reshape(1, _C))
    return out[:_N]


# speedup vs baseline: 16.4091x; 1.3545x over previous
import functools

import jax
import jax.numpy as jnp
from jax import lax
from jax.experimental import pallas as pl
from jax.experimental.pallas import tpu as pltpu
from jax.experimental.pallas import tpu_sc as plsc

_N = 10000
_E = 320000
_C = 128
_EPS = 1e-5
_NP = 10240
_B = 128
_NCHUNK = 2560
_EPAD = _NCHUNK * _B
_NC = 2
_NS = 16
_GT = _NCHUNK // (_NC * _NS)
_ZR = _NP // _NS
_DW = 16

_DR = _NP // _B


def _deg_body(row2d, out, accd, rowv, deg1, deg2, idv):
    c = lax.axis_index("c")
    s = lax.axis_index("s")

    def zfill(i, carry):
        deg1[pl.ds(i * 16, 16)] = jnp.zeros((16,), jnp.float32)
        return carry

    lax.fori_loop(0, _NP // 16, zfill, 0)
    for k in range(_DR // 16):
        idv[0, pl.ds(k * 16, 16)] = lax.iota(jnp.int32, 16) + (k * 16)

    def z2(i, carry):
        for j in range(_B // 16):
            deg2[i, pl.ds(j * 16, 16)] = jnp.zeros((16,), jnp.float32)
        return carry

    lax.fori_loop(0, 8, z2, 0)

    @pl.when(s < _DR // 8)
    def _zero_acc():
        pltpu.sync_copy(deg2.at[pl.ds(0, 8)], accd.at[pl.ds(s * 8, 8)])

    base = (c * _NS + s) * _GT
    pltpu.sync_copy(row2d.at[pl.ds(base, _GT)], rowv)
    plsc.subcore_barrier()

    ones16 = jnp.full((16,), 1.0, jnp.float32)

    def hist(g, carry):
        for j in range(_B // 16):
            idx = rowv[g, pl.ds(j * 16, 16)]
            plsc.addupdate_scatter(deg1, [idx], ones16)
        return carry

    lax.fori_loop(0, _GT, hist, 0)

    def tw(g, carry):
        for j in range(_B // 16):
            deg2[g, pl.ds(j * 16, 16)] = deg1[pl.ds(g * _B + j * 16, 16)]
        return carry

    lax.fori_loop(0, _DR, tw, 0)
    pltpu.sync_copy(deg2, accd.at[idv.at[0]], add=True)
    plsc.subcore_barrier()

    @pl.when(s < _DR // 8)
    def _dump():
        pltpu.sync_copy(accd.at[pl.ds(s * 8, 8)],
                        out.at[pl.ds(c * _DR + s * 8, 8)])


@functools.cache
def _deg_kernel():
    mesh = plsc.VectorSubcoreMesh(core_axis_name="c", subcore_axis_name="s")
    return pl.kernel(
        _deg_body,
        out_type=jax.ShapeDtypeStruct((_NC * _DR, _B), jnp.float32),
        mesh=mesh,
        compiler_params=pltpu.CompilerParams(needs_layout_passes=False),
        scratch_types=[
            pltpu.VMEM_SHARED((_DR, _B), jnp.float32),
            pltpu.VMEM((_GT, _B), jnp.int32),
            pltpu.VMEM((_NP,), jnp.float32),
            pltpu.VMEM((_DR, _B), jnp.float32),
            pltpu.VMEM((1, _DR), jnp.int32),
        ],
    )


def _prop_body(hs, col2d, row2d, out, acc, colv, rowv, gbuf, gsem):
    c = lax.axis_index("c")
    s = lax.axis_index("s")

    def fill(i, carry):
        for j in range(_C // 16):
            gbuf[i, pl.ds(j * 16, 16)] = jnp.zeros((16,), jnp.float32)
        return carry

    lax.fori_loop(0, _B, fill, 0)

    def zero(k, carry):
        pltpu.sync_copy(gbuf, acc.at[pl.ds(s * _ZR + k * _B, _B)])
        return carry

    lax.fori_loop(0, _ZR // _B, zero, 0)

    base = (c * _NS + s) * _GT
    pltpu.sync_copy(col2d.at[pl.ds(base, _GT)], colv)
    pltpu.sync_copy(row2d.at[pl.ds(base, _GT)], rowv)
    plsc.subcore_barrier()

    def chunk(g, carry):
        pltpu.async_copy(hs.at[colv.at[g]], gbuf, gsem).wait()
        pltpu.sync_copy(gbuf, acc.at[rowv.at[g]], add=True)
        return carry

    lax.fori_loop(0, _GT, chunk, 0)
    plsc.subcore_barrier()
    pltpu.sync_copy(acc.at[pl.ds(s * _ZR, _ZR)],
                    out.at[pl.ds(c * _NP + s * _ZR, _ZR)])


@functools.cache
def _prop_kernel():
    mesh = plsc.VectorSubcoreMesh(core_axis_name="c", subcore_axis_name="s")
    return pl.kernel(
        _prop_body,
        out_type=jax.ShapeDtypeStruct((_NC * _NP, _C), jnp.float32),
        mesh=mesh,
        scratch_types=[
            pltpu.VMEM_SHARED((_NP, _C), jnp.float32),
            pltpu.VMEM((_GT, _B), jnp.int32),
            pltpu.VMEM((_GT, _B), jnp.int32),
            pltpu.VMEM((_B, _C), jnp.float32),
            pltpu.SemaphoreType.DMA,
        ],
    )


def _tc1_body(degA, degB, x, dinv_o, hs1_o):
    d = degA[...] + degB[...]
    dinv = jnp.where(d > 0.0, lax.rsqrt(d), 0.0)
    dinv_o[...] = dinv
    hs1_o[...] = x[...] * dinv


def _tc1(degA, degB, x_pad):
    return pl.pallas_call(
        _tc1_body,
        out_shape=[
            jax.ShapeDtypeStruct((_NP, 1), jnp.float32),
            jax.ShapeDtypeStruct((_NP, _C), jnp.float32),
        ],
    )(degA, degB, x_pad)


def _tc2_body(S, dinv, x, Wm, bm, tx1_o, hs2_o, id_o):
    ssum = S[0] + S[1]
    d = dinv[...]
    tx1 = -(d * ssum)
    tx1_o[...] = tx1
    hs2_o[...] = d * tx1
    id_o[...] = jnp.dot(x[...], Wm[...],
                        preferred_element_type=jnp.float32) + bm[...]


def _tc2(S1, dinv, x_pad, W_match, b_match):
    return pl.pallas_call(
        _tc2_body,
        out_shape=[
            jax.ShapeDtypeStruct((_NP, _C), jnp.float32),
            jax.ShapeDtypeStruct((_NP, _C), jnp.float32),
            jax.ShapeDtypeStruct((_NP, _C), jnp.float32),
        ],
    )(S1, dinv, x_pad, W_match, b_match)


_RB = 1024
_NBLK = _NP // _RB


def _tc3_body(S, dinv, x, tx1, Wc, bc, raw_o, ps_o, pq_o):
    i = pl.program_id(0)
    ssum = S[0] + S[1]
    d = dinv[...]
    xv = x[...]
    tx1v = tx1[...]
    tx2 = -2.0 * (d * ssum) - xv
    raw = (jnp.dot(xv, Wc[0], preferred_element_type=jnp.float32)
           + jnp.dot(tx1v, Wc[1], preferred_element_type=jnp.float32)
           + jnp.dot(tx2, Wc[2], preferred_element_type=jnp.float32)
           + bc[...])
    rows = lax.broadcasted_iota(jnp.int32, (_RB, _C), 0) + i * _RB
    raw = jnp.where(rows < _N, raw, 0.0)
    raw_o[...] = raw
    ps_o[...] = jnp.sum(raw, axis=0, keepdims=True)[None]
    pq_o[...] = jnp.sum(raw * raw, axis=0, keepdims=True)[None]


def _tc3(S2, dinv, x_pad, tx1, W_cheb, b_cheb):
    return pl.pallas_call(
        _tc3_body,
        grid=(_NBLK,),
        in_specs=[
            pl.BlockSpec((2, _RB, _C), lambda i: (0, i, 0)),
            pl.BlockSpec((_RB, 1), lambda i: (i, 0)),
            pl.BlockSpec((_RB, _C), lambda i: (i, 0)),
            pl.BlockSpec((_RB, _C), lambda i: (i, 0)),
            pl.BlockSpec((3, _C, _C), lambda i: (0, 0, 0)),
            pl.BlockSpec((1, _C), lambda i: (0, 0)),
        ],
        out_specs=[
            pl.BlockSpec((_RB, _C), lambda i: (i, 0)),
            pl.BlockSpec((1, 1, _C), lambda i: (i, 0, 0)),
            pl.BlockSpec((1, 1, _C), lambda i: (i, 0, 0)),
        ],
        out_shape=[
            jax.ShapeDtypeStruct((_NP, _C), jnp.float32),
            jax.ShapeDtypeStruct((_NBLK, 1, _C), jnp.float32),
            jax.ShapeDtypeStruct((_NBLK, 1, _C), jnp.float32),
        ],
    )(S2, dinv, x_pad, tx1, W_cheb, b_cheb)


def _tc4_body(raw, ident, ps, pq, gamma, beta, out_o):
    mean = jnp.sum(ps[...], axis=0) * (1.0 / _N)
    var = jnp.sum(pq[...], axis=0) * (1.0 / _N) - mean * mean
    inv = lax.rsqrt(var + _EPS)
    y = (raw[...] - mean) * inv * gamma[...] + beta[...]
    out_o[...] = jnp.maximum(y, 0.0) + ident[...]


def _tc4(raw, ident, ps, pq, gamma, beta):
    return pl.pallas_call(
        _tc4_body,
        grid=(_NBLK,),
        in_specs=[
            pl.BlockSpec((_RB, _C), lambda i: (i, 0)),
            pl.BlockSpec((_RB, _C), lambda i: (i, 0)),
            pl.BlockSpec((_NBLK, 1, _C), lambda i: (0, 0, 0)),
            pl.BlockSpec((_NBLK, 1, _C), lambda i: (0, 0, 0)),
            pl.BlockSpec((1, _C), lambda i: (0, 0)),
            pl.BlockSpec((1, _C), lambda i: (0, 0)),
        ],
        out_specs=pl.BlockSpec((_RB, _C), lambda i: (i, 0)),
        out_shape=jax.ShapeDtypeStruct((_NP, _C), jnp.float32),
    )(raw, ident, ps, pq, gamma, beta)


def kernel(edge_index, x, W_cheb, b_cheb, gamma, beta, W_match, b_match):
    ei = edge_index.astype(jnp.int32)
    pad = _N + (jnp.arange(_EPAD - _E, dtype=jnp.int32) % (_NP - _N))
    row2d = jnp.concatenate([ei[0], pad]).reshape(_NCHUNK, _B)
    col2d = jnp.concatenate([ei[1], pad]).reshape(_NCHUNK, _B)
    x_pad = jnp.pad(x, ((0, _NP - _N), (0, 0)))

    degp = _deg_kernel()(row2d)
    dinv, hs1 = _tc1(degp[:_DR].reshape(_NP, 1),
                     degp[_DR:].reshape(_NP, 1), x_pad)
    S1 = _prop_kernel()(hs1, col2d, row2d).reshape(_NC, _NP, _C)
    tx1, hs2, ident = _tc2(S1, dinv, x_pad, W_match,
                           b_match.reshape(1, _C))
    S2 = _prop_kernel()(hs2, col2d, row2d).reshape(_NC, _NP, _C)
    raw, ps, pq = _tc3(S2, dinv, x_pad, tx1, W_cheb,
                       b_cheb.reshape(1, _C))
    out = _tc4(raw, ident, ps, pq, gamma.reshape(1, _C),
               beta.reshape(1, _C))
    return out[:_N]


# speedup vs baseline: 23.3027x; 1.4201x over previous
import functools

import jax
import jax.numpy as jnp
from jax import lax
from jax.experimental import pallas as pl
from jax.experimental.pallas import tpu as pltpu
from jax.experimental.pallas import tpu_sc as plsc

_N = 10000
_E = 320000
_C = 128
_EPS = 1e-5
_NP = 10240
_B = 128
_NCHUNK = 2560
_EPAD = _NCHUNK * _B
_NC = 2
_NS = 16
_GT = _NCHUNK // (_NC * _NS)
_GH = _GT // 2
_ZR = _NP // _NS
_DW = 16

_DR = _NP // _B


def _deg_body(row2d, out, accd, rowv, deg1, deg2, idv):
    c = lax.axis_index("c")
    s = lax.axis_index("s")

    def zfill(i, carry):
        deg1[pl.ds(i * 16, 16)] = jnp.zeros((16,), jnp.float32)
        return carry

    lax.fori_loop(0, _NP // 16, zfill, 0)
    for k in range(_DR // 16):
        idv[0, pl.ds(k * 16, 16)] = lax.iota(jnp.int32, 16) + (k * 16)

    def z2(i, carry):
        for j in range(_B // 16):
            deg2[i, pl.ds(j * 16, 16)] = jnp.zeros((16,), jnp.float32)
        return carry

    lax.fori_loop(0, 8, z2, 0)

    @pl.when(s < _DR // 8)
    def _zero_acc():
        pltpu.sync_copy(deg2.at[pl.ds(0, 8)], accd.at[pl.ds(s * 8, 8)])

    base = (c * _NS + s) * _GT
    pltpu.sync_copy(row2d.at[pl.ds(base, _GT)], rowv)
    plsc.subcore_barrier()

    ones16 = jnp.full((16,), 1.0, jnp.float32)

    def hist(g, carry):
        for j in range(_B // 16):
            idx = rowv[g, pl.ds(j * 16, 16)]
            plsc.addupdate_scatter(deg1, [idx], ones16)
        return carry

    lax.fori_loop(0, _GT, hist, 0)

    def tw(g, carry):
        for j in range(_B // 16):
            deg2[g, pl.ds(j * 16, 16)] = deg1[pl.ds(g * _B + j * 16, 16)]
        return carry

    lax.fori_loop(0, _DR, tw, 0)
    pltpu.sync_copy(deg2, accd.at[idv.at[0]], add=True)
    plsc.subcore_barrier()

    @pl.when(s < _DR // 8)
    def _dump():
        pltpu.sync_copy(accd.at[pl.ds(s * 8, 8)],
                        out.at[pl.ds(c * _DR + s * 8, 8)])


@functools.cache
def _deg_kernel():
    mesh = plsc.VectorSubcoreMesh(core_axis_name="c", subcore_axis_name="s")
    return pl.kernel(
        _deg_body,
        out_type=jax.ShapeDtypeStruct((_NC * _DR, _B), jnp.float32),
        mesh=mesh,
        compiler_params=pltpu.CompilerParams(needs_layout_passes=False),
        scratch_types=[
            pltpu.VMEM_SHARED((_DR, _B), jnp.float32),
            pltpu.VMEM((_GT, _B), jnp.int32),
            pltpu.VMEM((_NP,), jnp.float32),
            pltpu.VMEM((_DR, _B), jnp.float32),
            pltpu.VMEM((1, _DR), jnp.int32),
        ],
    )


def _prop_body(hs, col2d, row2d, out, acc, colv, rowv, gbuf0, gbuf1,
               gsem0, gsem1):
    c = lax.axis_index("c")
    s = lax.axis_index("s")

    def fill(i, carry):
        for j in range(_C // 16):
            gbuf0[i, pl.ds(j * 16, 16)] = jnp.zeros((16,), jnp.float32)
        return carry

    lax.fori_loop(0, _B, fill, 0)

    def zero(k, carry):
        pltpu.sync_copy(gbuf0, acc.at[pl.ds(s * _ZR + k * _B, _B)])
        return carry

    lax.fori_loop(0, _ZR // _B, zero, 0)

    base = (c * _NS + s) * _GT
    plsc.subcore_barrier()

    def half(hh, carry):
        hbase = base + hh * _GH
        pltpu.sync_copy(col2d.at[pl.ds(hbase, _GH)], colv)
        pltpu.sync_copy(row2d.at[pl.ds(hbase, _GH)], rowv)
        pltpu.async_copy(hs.at[colv.at[0]], gbuf0, gsem0)
        pltpu.async_copy(hs.at[colv.at[1]], gbuf1, gsem1)

        def chunk2(h, carry2):
            for (buf, sem, off) in ((gbuf0, gsem0, 0), (gbuf1, gsem1, 1)):
                g = h * 2 + off
                pltpu.make_async_copy(hs.at[colv.at[g]], buf, sem).wait()
                pltpu.sync_copy(buf, acc.at[rowv.at[g]], add=True)

                @pl.when(g + 2 < _GH)
                def _refill():
                    pltpu.async_copy(hs.at[colv.at[g + 2]], buf, sem)

            return carry2

        lax.fori_loop(0, _GH // 2, chunk2, 0)
        return carry

    lax.fori_loop(0, _GT // _GH, half, 0)
    plsc.subcore_barrier()
    pltpu.sync_copy(acc.at[pl.ds(s * _ZR, _ZR)],
                    out.at[pl.ds(c * _NP + s * _ZR, _ZR)])


@functools.cache
def _prop_kernel():
    mesh = plsc.VectorSubcoreMesh(core_axis_name="c", subcore_axis_name="s")
    return pl.kernel(
        _prop_body,
        out_type=jax.ShapeDtypeStruct((_NC * _NP, _C), jnp.float32),
        mesh=mesh,
        scratch_types=[
            pltpu.VMEM_SHARED((_NP, _C), jnp.float32),
            pltpu.VMEM((_GH, _B), jnp.int32),
            pltpu.VMEM((_GH, _B), jnp.int32),
            pltpu.VMEM((_B, _C), jnp.float32),
            pltpu.VMEM((_B, _C), jnp.float32),
            pltpu.SemaphoreType.DMA,
            pltpu.SemaphoreType.DMA,
        ],
    )


def _tc1_body(degA, degB, x, dinv_o, hs1_o):
    d = degA[...] + degB[...]
    dinv = jnp.where(d > 0.0, lax.rsqrt(d), 0.0)
    dinv_o[...] = dinv
    hs1_o[...] = x[...] * dinv


def _tc1(degA, degB, x_pad):
    return pl.pallas_call(
        _tc1_body,
        out_shape=[
            jax.ShapeDtypeStruct((_NP, 1), jnp.float32),
            jax.ShapeDtypeStruct((_NP, _C), jnp.float32),
        ],
    )(degA, degB, x_pad)


def _tc2_body(S, dinv, x, Wm, bm, tx1_o, hs2_o, id_o):
    ssum = S[0] + S[1]
    d = dinv[...]
    tx1 = -(d * ssum)
    tx1_o[...] = tx1
    hs2_o[...] = d * tx1
    id_o[...] = jnp.dot(x[...], Wm[...],
                        preferred_element_type=jnp.float32) + bm[...]


def _tc2(S1, dinv, x_pad, W_match, b_match):
    return pl.pallas_call(
        _tc2_body,
        out_shape=[
            jax.ShapeDtypeStruct((_NP, _C), jnp.float32),
            jax.ShapeDtypeStruct((_NP, _C), jnp.float32),
            jax.ShapeDtypeStruct((_NP, _C), jnp.float32),
        ],
    )(S1, dinv, x_pad, W_match, b_match)


_RB = 1024
_NBLK = _NP // _RB


def _tc3_body(S, dinv, x, tx1, Wc, bc, raw_o, ps_o, pq_o):
    i = pl.program_id(0)
    ssum = S[0] + S[1]
    d = dinv[...]
    xv = x[...]
    tx1v = tx1[...]
    tx2 = -2.0 * (d * ssum) - xv
    raw = (jnp.dot(xv, Wc[0], preferred_element_type=jnp.float32)
           + jnp.dot(tx1v, Wc[1], preferred_element_type=jnp.float32)
           + jnp.dot(tx2, Wc[2], preferred_element_type=jnp.float32)
           + bc[...])
    rows = lax.broadcasted_iota(jnp.int32, (_RB, _C), 0) + i * _RB
    raw = jnp.where(rows < _N, raw, 0.0)
    raw_o[...] = raw
    ps_o[...] = jnp.sum(raw, axis=0, keepdims=True)[None]
    pq_o[...] = jnp.sum(raw * raw, axis=0, keepdims=True)[None]


def _tc3(S2, dinv, x_pad, tx1, W_cheb, b_cheb):
    return pl.pallas_call(
        _tc3_body,
        grid=(_NBLK,),
        in_specs=[
            pl.BlockSpec((2, _RB, _C), lambda i: (0, i, 0)),
            pl.BlockSpec((_RB, 1), lambda i: (i, 0)),
            pl.BlockSpec((_RB, _C), lambda i: (i, 0)),
            pl.BlockSpec((_RB, _C), lambda i: (i, 0)),
            pl.BlockSpec((3, _C, _C), lambda i: (0, 0, 0)),
            pl.BlockSpec((1, _C), lambda i: (0, 0)),
        ],
        out_specs=[
            pl.BlockSpec((_RB, _C), lambda i: (i, 0)),
            pl.BlockSpec((1, 1, _C), lambda i: (i, 0, 0)),
            pl.BlockSpec((1, 1, _C), lambda i: (i, 0, 0)),
        ],
        out_shape=[
            jax.ShapeDtypeStruct((_NP, _C), jnp.float32),
            jax.ShapeDtypeStruct((_NBLK, 1, _C), jnp.float32),
            jax.ShapeDtypeStruct((_NBLK, 1, _C), jnp.float32),
        ],
    )(S2, dinv, x_pad, tx1, W_cheb, b_cheb)


def _tc4_body(raw, ident, ps, pq, gamma, beta, out_o):
    mean = jnp.sum(ps[...], axis=0) * (1.0 / _N)
    var = jnp.sum(pq[...], axis=0) * (1.0 / _N) - mean * mean
    inv = lax.rsqrt(var + _EPS)
    y = (raw[...] - mean) * inv * gamma[...] + beta[...]
    out_o[...] = jnp.maximum(y, 0.0) + ident[...]


def _tc4(raw, ident, ps, pq, gamma, beta):
    return pl.pallas_call(
        _tc4_body,
        grid=(_NBLK,),
        in_specs=[
            pl.BlockSpec((_RB, _C), lambda i: (i, 0)),
            pl.BlockSpec((_RB, _C), lambda i: (i, 0)),
            pl.BlockSpec((_NBLK, 1, _C), lambda i: (0, 0, 0)),
            pl.BlockSpec((_NBLK, 1, _C), lambda i: (0, 0, 0)),
            pl.BlockSpec((1, _C), lambda i: (0, 0)),
            pl.BlockSpec((1, _C), lambda i: (0, 0)),
        ],
        out_specs=pl.BlockSpec((_RB, _C), lambda i: (i, 0)),
        out_shape=jax.ShapeDtypeStruct((_NP, _C), jnp.float32),
    )(raw, ident, ps, pq, gamma, beta)


def kernel(edge_index, x, W_cheb, b_cheb, gamma, beta, W_match, b_match):
    ei = edge_index.astype(jnp.int32)
    pad = _N + (jnp.arange(_EPAD - _E, dtype=jnp.int32) % (_NP - _N))
    row2d = jnp.concatenate([ei[0], pad]).reshape(_NCHUNK, _B)
    col2d = jnp.concatenate([ei[1], pad]).reshape(_NCHUNK, _B)
    x_pad = jnp.pad(x, ((0, _NP - _N), (0, 0)))

    degp = _deg_kernel()(row2d)
    dinv, hs1 = _tc1(degp[:_DR].reshape(_NP, 1),
                     degp[_DR:].reshape(_NP, 1), x_pad)
    S1 = _prop_kernel()(hs1, col2d, row2d).reshape(_NC, _NP, _C)
    tx1, hs2, ident = _tc2(S1, dinv, x_pad, W_match,
                           b_match.reshape(1, _C))
    S2 = _prop_kernel()(hs2, col2d, row2d).reshape(_NC, _NP, _C)
    raw, ps, pq = _tc3(S2, dinv, x_pad, tx1, W_cheb,
                       b_cheb.reshape(1, _C))
    out = _tc4(raw, ident, ps, pq, gamma.reshape(1, _C),
               beta.reshape(1, _C))
    return out[:_N]


# speedup vs baseline: 24.2141x; 1.0391x over previous
import functools

import jax
import jax.numpy as jnp
from jax import lax
from jax.experimental import pallas as pl
from jax.experimental.pallas import tpu as pltpu
from jax.experimental.pallas import tpu_sc as plsc

_N = 10000
_E = 320000
_C = 128
_EPS = 1e-5
_NP = 10240
_B = 128
_NCHUNK = 2560
_EPAD = _NCHUNK * _B
_NC = 2
_NS = 16
_GT = _NCHUNK // (_NC * _NS)
_GH = _GT // 2
_ZR = _NP // _NS
_DW = 16

_DR = _NP // _B


def _deg_body(row2d, out, accd, rowv, deg1, deg2, idv):
    c = lax.axis_index("c")
    s = lax.axis_index("s")

    def zfill(i, carry):
        deg1[pl.ds(i * 16, 16)] = jnp.zeros((16,), jnp.float32)
        return carry

    lax.fori_loop(0, _NP // 16, zfill, 0)
    for k in range(_DR // 16):
        idv[0, pl.ds(k * 16, 16)] = lax.iota(jnp.int32, 16) + (k * 16)

    def z2(i, carry):
        for j in range(_B // 16):
            deg2[i, pl.ds(j * 16, 16)] = jnp.zeros((16,), jnp.float32)
        return carry

    lax.fori_loop(0, 8, z2, 0)

    @pl.when(s < _DR // 8)
    def _zero_acc():
        pltpu.sync_copy(deg2.at[pl.ds(0, 8)], accd.at[pl.ds(s * 8, 8)])

    base = (c * _NS + s) * _GT
    pltpu.sync_copy(row2d.at[pl.ds(base, _GT)], rowv)
    plsc.subcore_barrier()

    ones16 = jnp.full((16,), 1.0, jnp.float32)

    def hist(g, carry):
        for j in range(_B // 16):
            idx = rowv[g, pl.ds(j * 16, 16)]
            plsc.addupdate_scatter(deg1, [idx], ones16)
        return carry

    lax.fori_loop(0, _GT, hist, 0)

    def tw(g, carry):
        for j in range(_B // 16):
            deg2[g, pl.ds(j * 16, 16)] = deg1[pl.ds(g * _B + j * 16, 16)]
        return carry

    lax.fori_loop(0, _DR, tw, 0)
    pltpu.sync_copy(deg2, accd.at[idv.at[0]], add=True)
    plsc.subcore_barrier()

    @pl.when(s < _DR // 8)
    def _dump():
        pltpu.sync_copy(accd.at[pl.ds(s * 8, 8)],
                        out.at[pl.ds(c * _DR + s * 8, 8)])


@functools.cache
def _deg_kernel():
    mesh = plsc.VectorSubcoreMesh(core_axis_name="c", subcore_axis_name="s")
    return pl.kernel(
        _deg_body,
        out_type=jax.ShapeDtypeStruct((_NC * _DR, _B), jnp.float32),
        mesh=mesh,
        compiler_params=pltpu.CompilerParams(needs_layout_passes=False),
        scratch_types=[
            pltpu.VMEM_SHARED((_DR, _B), jnp.float32),
            pltpu.VMEM((_GT, _B), jnp.int32),
            pltpu.VMEM((_NP,), jnp.float32),
            pltpu.VMEM((_DR, _B), jnp.float32),
            pltpu.VMEM((1, _DR), jnp.int32),
        ],
    )


def _prop_body(hs, col2d, row2d, out, acc, colv, rowv, gbuf0, gbuf1,
               gsem0, gsem1):
    c = lax.axis_index("c")
    s = lax.axis_index("s")

    def fill(i, carry):
        for j in range(_C // 16):
            gbuf0[i, pl.ds(j * 16, 16)] = jnp.zeros((16,), jnp.float32)
        return carry

    lax.fori_loop(0, _B, fill, 0)

    def zero(k, carry):
        pltpu.sync_copy(gbuf0, acc.at[pl.ds(s * _ZR + k * _B, _B)])
        return carry

    lax.fori_loop(0, _ZR // _B, zero, 0)

    base = (c * _NS + s) * _GT
    plsc.subcore_barrier()

    def half(hh, carry):
        hbase = base + hh * _GH
        pltpu.sync_copy(col2d.at[pl.ds(hbase, _GH)], colv)
        pltpu.sync_copy(row2d.at[pl.ds(hbase, _GH)], rowv)
        pltpu.async_copy(hs.at[colv.at[0]], gbuf0, gsem0)
        pltpu.async_copy(hs.at[colv.at[1]], gbuf1, gsem1)

        def chunk2(h, carry2):
            for (buf, sem, off) in ((gbuf0, gsem0, 0), (gbuf1, gsem1, 1)):
                g = h * 2 + off
                pltpu.make_async_copy(hs.at[colv.at[g]], buf, sem).wait()
                pltpu.sync_copy(buf, acc.at[rowv.at[g]], add=True)

                @pl.when(g + 2 < _GH)
                def _refill():
                    pltpu.async_copy(hs.at[colv.at[g + 2]], buf, sem)

            return carry2

        lax.fori_loop(0, _GH // 2, chunk2, 0)
        return carry

    lax.fori_loop(0, _GT // _GH, half, 0)
    plsc.subcore_barrier()
    pltpu.sync_copy(acc.at[pl.ds(s * _ZR, _ZR)],
                    out.at[pl.ds(c * _NP + s * _ZR, _ZR)])


@functools.cache
def _prop_kernel():
    mesh = plsc.VectorSubcoreMesh(core_axis_name="c", subcore_axis_name="s")
    return pl.kernel(
        _prop_body,
        out_type=jax.ShapeDtypeStruct((_NC * _NP, _C), jnp.float32),
        mesh=mesh,
        scratch_types=[
            pltpu.VMEM_SHARED((_NP, _C), jnp.float32),
            pltpu.VMEM((_GH, _B), jnp.int32),
            pltpu.VMEM((_GH, _B), jnp.int32),
            pltpu.VMEM((_B, _C), jnp.float32),
            pltpu.VMEM((_B, _C), jnp.float32),
            pltpu.SemaphoreType.DMA,
            pltpu.SemaphoreType.DMA,
        ],
    )


def _tc1_body(degA, degB, x, dinv_o, hs1_o):
    d = degA[...] + degB[...]
    dinv = jnp.where(d > 0.0, lax.rsqrt(d), 0.0)
    dinv_o[...] = dinv
    hs1_o[...] = x[...] * dinv


def _tc1(degA, degB, x_pad):
    return pl.pallas_call(
        _tc1_body,
        out_shape=[
            jax.ShapeDtypeStruct((_NP, 1), jnp.float32),
            jax.ShapeDtypeStruct((_NP, _C), jnp.float32),
        ],
    )(degA, degB, x_pad)


def _tcid_body(x, Wm, bm, id_o):
    id_o[...] = jnp.dot(x[...], Wm[...],
                        preferred_element_type=jnp.float32) + bm[...]


def _tcid(x, W_match, b_match):
    return pl.pallas_call(
        _tcid_body,
        out_shape=jax.ShapeDtypeStruct((_N, _C), jnp.float32),
    )(x, W_match, b_match)


def _tc2_body(S, dinv, x, tx1_o, hs2_o):
    ssum = S[0] + S[1]
    d = dinv[...]
    tx1 = -(d * ssum)
    tx1_o[...] = tx1
    hs2_o[...] = d * tx1


def _tc2(S1, dinv, x_pad):
    return pl.pallas_call(
        _tc2_body,
        out_shape=[
            jax.ShapeDtypeStruct((_NP, _C), jnp.float32),
            jax.ShapeDtypeStruct((_NP, _C), jnp.float32),
        ],
    )(S1, dinv, x_pad)


def _tc3_body(S, dinv, x, tx1, Wc, bc, ident, gamma, beta, out_o):
    ssum = S[0] + S[1]
    d = dinv[...]
    xv = x[...]
    tx1v = tx1[...]
    tx2 = -2.0 * (d * ssum) - xv
    raw = (jnp.dot(xv, Wc[0], preferred_element_type=jnp.float32)
           + jnp.dot(tx1v, Wc[1], preferred_element_type=jnp.float32)
           + jnp.dot(tx2, Wc[2], preferred_element_type=jnp.float32)
           + bc[...])
    rows = lax.broadcasted_iota(jnp.int32, (_NP, _C), 0)
    raw = jnp.where(rows < _N, raw, 0.0)
    mean = jnp.sum(raw, axis=0, keepdims=True) * (1.0 / _N)
    var = jnp.sum(raw * raw, axis=0, keepdims=True) * (1.0 / _N) - mean * mean
    inv = lax.rsqrt(var + _EPS)
    y = (raw - mean) * inv * gamma[...] + beta[...]
    y = jnp.maximum(y, 0.0)[:_N] + ident[...]
    out_o[...] = y


def _tc3(S2, dinv, x_pad, tx1, W_cheb, b_cheb, ident, gamma, beta):
    return pl.pallas_call(
        _tc3_body,
        out_shape=jax.ShapeDtypeStruct((_N, _C), jnp.float32),
        compiler_params=pltpu.CompilerParams(
            vmem_limit_bytes=100 * 1024 * 1024),
    )(S2, dinv, x_pad, tx1, W_cheb, b_cheb, ident, gamma, beta)


def kernel(edge_index, x, W_cheb, b_cheb, gamma, beta, W_match, b_match):
    ei = edge_index.astype(jnp.int32)
    pad = _N + (jnp.arange(_EPAD - _E, dtype=jnp.int32) % (_NP - _N))
    row2d = jnp.concatenate([ei[0], pad]).reshape(_NCHUNK, _B)
    col2d = jnp.concatenate([ei[1], pad]).reshape(_NCHUNK, _B)
    x_pad = jnp.pad(x, ((0, _NP - _N), (0, 0)))

    ident = _tcid(x, W_match, b_match.reshape(1, _C))
    degp = _deg_kernel()(row2d)
    dinv, hs1 = _tc1(degp[:_DR].reshape(_NP, 1),
                     degp[_DR:].reshape(_NP, 1), x_pad)
    S1 = _prop_kernel()(hs1, col2d, row2d).reshape(_NC, _NP, _C)
    tx1, hs2 = _tc2(S1, dinv, x_pad)
    S2 = _prop_kernel()(hs2, col2d, row2d).reshape(_NC, _NP, _C)
    return _tc3(S2, dinv, x_pad, tx1, W_cheb, b_cheb.reshape(1, _C),
                ident, gamma.reshape(1, _C), beta.reshape(1, _C))


# speedup vs baseline: 24.6083x; 1.0163x over previous
import functools

import jax
import jax.numpy as jnp
from jax import lax
from jax.experimental import pallas as pl
from jax.experimental.pallas import tpu as pltpu
from jax.experimental.pallas import tpu_sc as plsc

_N = 10000
_E = 320000
_C = 128
_EPS = 1e-5
_NP = 10240
_B = 128
_NCHUNK = 2560
_EPAD = _NCHUNK * _B
_NC = 2
_NS = 16
_GT = _NCHUNK // (_NC * _NS)
_GH = _GT // 2
_ZR = _NP // _NS
_DW = 16

_DR = _NP // _B


def _deg_body(row2d, out, accd, rowv, deg1, deg2, idv):
    c = lax.axis_index("c")
    s = lax.axis_index("s")

    def zfill(i, carry):
        deg1[pl.ds(i * 16, 16)] = jnp.zeros((16,), jnp.float32)
        return carry

    lax.fori_loop(0, _NP // 16, zfill, 0)
    for k in range(_DR // 16):
        idv[0, pl.ds(k * 16, 16)] = lax.iota(jnp.int32, 16) + (k * 16)

    def z2(i, carry):
        for j in range(_B // 16):
            deg2[i, pl.ds(j * 16, 16)] = jnp.zeros((16,), jnp.float32)
        return carry

    lax.fori_loop(0, 8, z2, 0)

    @pl.when(s < _DR // 8)
    def _zero_acc():
        pltpu.sync_copy(deg2.at[pl.ds(0, 8)], accd.at[pl.ds(s * 8, 8)])

    base = (c * _NS + s) * _GT
    pltpu.sync_copy(row2d.at[pl.ds(base, _GT)], rowv)
    plsc.subcore_barrier()

    ones16 = jnp.full((16,), 1.0, jnp.float32)

    def hist(g, carry):
        for j in range(_B // 16):
            idx = rowv[g, pl.ds(j * 16, 16)]
            plsc.addupdate_scatter(deg1, [idx], ones16)
        return carry

    lax.fori_loop(0, _GT, hist, 0)

    def tw(g, carry):
        for j in range(_B // 16):
            deg2[g, pl.ds(j * 16, 16)] = deg1[pl.ds(g * _B + j * 16, 16)]
        return carry

    lax.fori_loop(0, _DR, tw, 0)
    pltpu.sync_copy(deg2, accd.at[idv.at[0]], add=True)
    plsc.subcore_barrier()

    @pl.when(s < _DR // 8)
    def _dump():
        pltpu.sync_copy(accd.at[pl.ds(s * 8, 8)],
                        out.at[pl.ds(c * _DR + s * 8, 8)])


@functools.cache
def _deg_kernel():
    mesh = plsc.VectorSubcoreMesh(core_axis_name="c", subcore_axis_name="s")
    return pl.kernel(
        _deg_body,
        out_type=jax.ShapeDtypeStruct((_NC * _DR, _B), jnp.float32),
        mesh=mesh,
        compiler_params=pltpu.CompilerParams(needs_layout_passes=False),
        scratch_types=[
            pltpu.VMEM_SHARED((_DR, _B), jnp.float32),
            pltpu.VMEM((_GT, _B), jnp.int32),
            pltpu.VMEM((_NP,), jnp.float32),
            pltpu.VMEM((_DR, _B), jnp.float32),
            pltpu.VMEM((1, _DR), jnp.int32),
        ],
    )


def _prop_body(hs, col2d, row2d, out, acc, colv, rowv, gbuf0, gbuf1,
               gsem0, gsem1):
    c = lax.axis_index("c")
    s = lax.axis_index("s")

    def fill(i, carry):
        for j in range(_C // 16):
            gbuf0[i, pl.ds(j * 16, 16)] = jnp.zeros((16,), jnp.float32)
        return carry

    lax.fori_loop(0, _B, fill, 0)

    def zero(k, carry):
        pltpu.sync_copy(gbuf0, acc.at[pl.ds(s * _ZR + k * _B, _B)])
        return carry

    lax.fori_loop(0, _ZR // _B, zero, 0)

    base = (c * _NS + s) * _GT
    plsc.subcore_barrier()

    def half(hh, carry):
        hbase = base + hh * _GH
        pltpu.sync_copy(col2d.at[pl.ds(hbase, _GH)], colv)
        pltpu.sync_copy(row2d.at[pl.ds(hbase, _GH)], rowv)
        pltpu.async_copy(hs.at[colv.at[0]], gbuf0, gsem0)
        pltpu.async_copy(hs.at[colv.at[1]], gbuf1, gsem1)

        def chunk2(h, carry2):
            for (buf, sem, off) in ((gbuf0, gsem0, 0), (gbuf1, gsem1, 1)):
                g = h * 2 + off
                pltpu.make_async_copy(hs.at[colv.at[g]], buf, sem).wait()
                pltpu.sync_copy(buf, acc.at[rowv.at[g]], add=True)

                @pl.when(g + 2 < _GH)
                def _refill():
                    pltpu.async_copy(hs.at[colv.at[g + 2]], buf, sem)

            return carry2

        lax.fori_loop(0, _GH // 2, chunk2, 0)
        return carry

    lax.fori_loop(0, _GT // _GH, half, 0)
    plsc.subcore_barrier()
    pltpu.sync_copy(acc.at[pl.ds(s * _ZR, _ZR)],
                    out.at[pl.ds(c * _NP + s * _ZR, _ZR)])


@functools.cache
def _prop_kernel():
    mesh = plsc.VectorSubcoreMesh(core_axis_name="c", subcore_axis_name="s")
    return pl.kernel(
        _prop_body,
        out_type=jax.ShapeDtypeStruct((_NC * _NP, _C), jnp.float32),
        mesh=mesh,
        scratch_types=[
            pltpu.VMEM_SHARED((_NP, _C), jnp.float32),
            pltpu.VMEM((_GH, _B), jnp.int32),
            pltpu.VMEM((_GH, _B), jnp.int32),
            pltpu.VMEM((_B, _C), jnp.float32),
            pltpu.VMEM((_B, _C), jnp.float32),
            pltpu.SemaphoreType.DMA,
            pltpu.SemaphoreType.DMA,
        ],
    )


def _tc1_body(degA, degB, x, dinv_o, hs1_o):
    d = degA[...] + degB[...]
    dinv = jnp.where(d > 0.0, lax.rsqrt(d), 0.0)
    dinv_o[...] = dinv
    hs1_o[pl.ds(0, _N), :] = x[...] * dinv[:_N]


def _tc1(degA, degB, x):
    return pl.pallas_call(
        _tc1_body,
        out_shape=[
            jax.ShapeDtypeStruct((_NP, 1), jnp.float32),
            jax.ShapeDtypeStruct((_NP, _C), jnp.float32),
        ],
    )(degA, degB, x)


def _tcid_body(x, Wm, bm, id_o):
    id_o[...] = jnp.dot(x[...], Wm[...],
                        preferred_element_type=jnp.float32) + bm[...]


def _tcid(x, W_match, b_match):
    return pl.pallas_call(
        _tcid_body,
        out_shape=jax.ShapeDtypeStruct((_N, _C), jnp.float32),
    )(x, W_match, b_match)


def _tc2_body(S, dinv, hs2_o):
    ssum = S[0, pl.ds(0, _N), :] + S[1, pl.ds(0, _N), :]
    d = dinv[pl.ds(0, _N)]
    hs2_o[pl.ds(0, _N), :] = -(d * d * ssum)


def _tc2(S1, dinv):
    return pl.pallas_call(
        _tc2_body,
        out_shape=jax.ShapeDtypeStruct((_NP, _C), jnp.float32),
    )(S1, dinv)


def _tc3_body(S, dinv, x, hs2, Wc, bc, ident, gamma, beta, out_o):
    ssum = S[0, pl.ds(0, _N), :] + S[1, pl.ds(0, _N), :]
    d = dinv[pl.ds(0, _N)]
    xv = x[...]
    tx1v = hs2[pl.ds(0, _N), :] * jnp.where(d > 0.0, 1.0 / d, 0.0)
    tx2 = -2.0 * (d * ssum) - xv
    raw = (jnp.dot(xv, Wc[0], preferred_element_type=jnp.float32)
           + jnp.dot(tx1v, Wc[1], preferred_element_type=jnp.float32)
           + jnp.dot(tx2, Wc[2], preferred_element_type=jnp.float32)
           + bc[...])
    mean = jnp.sum(raw, axis=0, keepdims=True) * (1.0 / _N)
    var = jnp.sum(raw * raw, axis=0, keepdims=True) * (1.0 / _N) - mean * mean
    inv = lax.rsqrt(var + _EPS)
    y = (raw - mean) * inv * gamma[...] + beta[...]
    out_o[...] = jnp.maximum(y, 0.0) + ident[...]


def _tc3(S2, dinv, x, hs2, W_cheb, b_cheb, ident, gamma, beta):
    return pl.pallas_call(
        _tc3_body,
        out_shape=jax.ShapeDtypeStruct((_N, _C), jnp.float32),
        compiler_params=pltpu.CompilerParams(
            vmem_limit_bytes=100 * 1024 * 1024),
    )(S2, dinv, x, hs2, W_cheb, b_cheb, ident, gamma, beta)


def kernel(edge_index, x, W_cheb, b_cheb, gamma, beta, W_match, b_match):
    ei = edge_index.astype(jnp.int32)
    pad = _N + (jnp.arange(_EPAD - _E, dtype=jnp.int32) % (_NP - _N))
    row2d = jnp.concatenate([ei[0], pad]).reshape(_NCHUNK, _B)
    col2d = jnp.concatenate([ei[1], pad]).reshape(_NCHUNK, _B)

    ident = _tcid(x, W_match, b_match.reshape(1, _C))
    degp = _deg_kernel()(row2d)
    dinv, hs1 = _tc1(degp[:_DR].reshape(_NP, 1),
                     degp[_DR:].reshape(_NP, 1), x)
    S1 = _prop_kernel()(hs1, col2d, row2d).reshape(_NC, _NP, _C)
    hs2 = _tc2(S1, dinv)
    S2 = _prop_kernel()(hs2, col2d, row2d).reshape(_NC, _NP, _C)
    return _tc3(S2, dinv, x, hs2, W_cheb, b_cheb.reshape(1, _C),
                ident, gamma.reshape(1, _C), beta.reshape(1, _C))


# speedup vs baseline: 24.7745x; 1.0068x over previous
import functools

import jax
import jax.numpy as jnp
from jax import lax
from jax.experimental import pallas as pl
from jax.experimental.pallas import tpu as pltpu
from jax.experimental.pallas import tpu_sc as plsc

_N = 10000
_E = 320000
_C = 128
_EPS = 1e-5
_NP = 10240
_B = 128
_NCHUNK = 2560
_EPAD = _NCHUNK * _B
_NC = 2
_NS = 16
_GT = _NCHUNK // (_NC * _NS)
_GH = _GT // 2
_ZR = _NP // _NS
_DW = 16

_DR = _NP // _B


def _deg_body(row2d, out, accd, rowv, deg1, deg2, idv):
    c = lax.axis_index("c")
    s = lax.axis_index("s")

    def zfill(i, carry):
        deg1[pl.ds(i * 16, 16)] = jnp.zeros((16,), jnp.float32)
        return carry

    lax.fori_loop(0, _NP // 16, zfill, 0)
    for k in range(_DR // 16):
        idv[0, pl.ds(k * 16, 16)] = lax.iota(jnp.int32, 16) + (k * 16)

    def z2(i, carry):
        for j in range(_B // 16):
            deg2[i, pl.ds(j * 16, 16)] = jnp.zeros((16,), jnp.float32)
        return carry

    lax.fori_loop(0, 8, z2, 0)

    @pl.when(s < _DR // 8)
    def _zero_acc():
        pltpu.sync_copy(deg2.at[pl.ds(0, 8)], accd.at[pl.ds(s * 8, 8)])

    base = (c * _NS + s) * _GT
    pltpu.sync_copy(row2d.at[pl.ds(base, _GT)], rowv)
    plsc.subcore_barrier()

    ones16 = jnp.full((16,), 1.0, jnp.float32)

    def hist(g, carry):
        for j in range(_B // 16):
            idx = rowv[g, pl.ds(j * 16, 16)]
            plsc.addupdate_scatter(deg1, [idx], ones16)
        return carry

    lax.fori_loop(0, _GT, hist, 0)

    def tw(g, carry):
        for j in range(_B // 16):
            deg2[g, pl.ds(j * 16, 16)] = deg1[pl.ds(g * _B + j * 16, 16)]
        return carry

    lax.fori_loop(0, _DR, tw, 0)
    pltpu.sync_copy(deg2, accd.at[idv.at[0]], add=True)
    plsc.subcore_barrier()

    @pl.when(s < _DR // 8)
    def _dump():
        pltpu.sync_copy(accd.at[pl.ds(s * 8, 8)],
                        out.at[pl.ds(c * _DR + s * 8, 8)])


@functools.cache
def _deg_kernel():
    mesh = plsc.VectorSubcoreMesh(core_axis_name="c", subcore_axis_name="s")
    return pl.kernel(
        _deg_body,
        out_type=jax.ShapeDtypeStruct((_NC * _DR, _B), jnp.float32),
        mesh=mesh,
        compiler_params=pltpu.CompilerParams(needs_layout_passes=False),
        scratch_types=[
            pltpu.VMEM_SHARED((_DR, _B), jnp.float32),
            pltpu.VMEM((_GT, _B), jnp.int32),
            pltpu.VMEM((_NP,), jnp.float32),
            pltpu.VMEM((_DR, _B), jnp.float32),
            pltpu.VMEM((1, _DR), jnp.int32),
        ],
    )


_ZB = 32


def _prop_body(hs, col2d, row2d, out, acc, colv, rowv, gbuf0, gbuf1, zbuf,
               gsem0, gsem1):
    c = lax.axis_index("c")
    s = lax.axis_index("s")
    base = (c * _NS + s) * _GT

    pltpu.sync_copy(col2d.at[pl.ds(base, _GH)], colv)
    pltpu.sync_copy(row2d.at[pl.ds(base, _GH)], rowv)
    pltpu.async_copy(hs.at[colv.at[0]], gbuf0, gsem0)
    pltpu.async_copy(hs.at[colv.at[1]], gbuf1, gsem1)

    def fill(i, carry):
        for j in range(_C // 16):
            zbuf[i, pl.ds(j * 16, 16)] = jnp.zeros((16,), jnp.float32)
        return carry

    lax.fori_loop(0, _ZB, fill, 0)

    def zero(k, carry):
        pltpu.sync_copy(zbuf, acc.at[pl.ds(s * _ZR + k * _ZB, _ZB)])
        return carry

    lax.fori_loop(0, _ZR // _ZB, zero, 0)
    plsc.subcore_barrier()

    def half(hh, carry):
        hbase = base + hh * _GH

        @pl.when(hh > 0)
        def _stage():
            pltpu.sync_copy(col2d.at[pl.ds(hbase, _GH)], colv)
            pltpu.sync_copy(row2d.at[pl.ds(hbase, _GH)], rowv)
            pltpu.async_copy(hs.at[colv.at[0]], gbuf0, gsem0)
            pltpu.async_copy(hs.at[colv.at[1]], gbuf1, gsem1)

        def chunk2(h, carry2):
            for (buf, sem, off) in ((gbuf0, gsem0, 0), (gbuf1, gsem1, 1)):
                g = h * 2 + off
                pltpu.make_async_copy(hs.at[colv.at[g]], buf, sem).wait()
                pltpu.sync_copy(buf, acc.at[rowv.at[g]], add=True)

                @pl.when(g + 2 < _GH)
                def _refill():
                    pltpu.async_copy(hs.at[colv.at[g + 2]], buf, sem)

            return carry2

        lax.fori_loop(0, _GH // 2, chunk2, 0)
        return carry

    lax.fori_loop(0, _GT // _GH, half, 0)
    plsc.subcore_barrier()
    pltpu.sync_copy(acc.at[pl.ds(s * _ZR, _ZR)],
                    out.at[pl.ds(c * _NP + s * _ZR, _ZR)])


@functools.cache
def _prop_kernel():
    mesh = plsc.VectorSubcoreMesh(core_axis_name="c", subcore_axis_name="s")
    return pl.kernel(
        _prop_body,
        out_type=jax.ShapeDtypeStruct((_NC * _NP, _C), jnp.float32),
        mesh=mesh,
        scratch_types=[
            pltpu.VMEM_SHARED((_NP, _C), jnp.float32),
            pltpu.VMEM((_GH, _B), jnp.int32),
            pltpu.VMEM((_GH, _B), jnp.int32),
            pltpu.VMEM((_B, _C), jnp.float32),
            pltpu.VMEM((_B, _C), jnp.float32),
            pltpu.VMEM((_ZB, _C), jnp.float32),
            pltpu.SemaphoreType.DMA,
            pltpu.SemaphoreType.DMA,
        ],
    )


def _tc1_body(degA, degB, x, dinv_o, hs1_o):
    d = degA[...] + degB[...]
    dinv = jnp.where(d > 0.0, lax.rsqrt(d), 0.0)
    dinv_o[...] = dinv
    hs1_o[pl.ds(0, _N), :] = x[...] * dinv[:_N]


def _tc1(degA, degB, x):
    return pl.pallas_call(
        _tc1_body,
        out_shape=[
            jax.ShapeDtypeStruct((_NP, 1), jnp.float32),
            jax.ShapeDtypeStruct((_NP, _C), jnp.float32),
        ],
    )(degA, degB, x)


def _tcid_body(x, Wm, bm, id_o):
    id_o[...] = jnp.dot(x[...], Wm[...],
                        preferred_element_type=jnp.float32) + bm[...]


def _tcid(x, W_match, b_match):
    return pl.pallas_call(
        _tcid_body,
        out_shape=jax.ShapeDtypeStruct((_N, _C), jnp.float32),
    )(x, W_match, b_match)


def _tc2_body(S, dinv, hs2_o):
    ssum = S[0, pl.ds(0, _N), :] + S[1, pl.ds(0, _N), :]
    d = dinv[pl.ds(0, _N)]
    hs2_o[pl.ds(0, _N), :] = -(d * d * ssum)


def _tc2(S1, dinv):
    return pl.pallas_call(
        _tc2_body,
        out_shape=jax.ShapeDtypeStruct((_NP, _C), jnp.float32),
    )(S1, dinv)


def _tc3_body(S, dinv, x, hs2, Wc, bc, ident, gamma, beta, out_o):
    ssum = S[0, pl.ds(0, _N), :] + S[1, pl.ds(0, _N), :]
    d = dinv[pl.ds(0, _N)]
    xv = x[...]
    tx1v = hs2[pl.ds(0, _N), :] * jnp.where(d > 0.0, 1.0 / d, 0.0)
    tx2 = -2.0 * (d * ssum) - xv
    raw = (jnp.dot(xv, Wc[0], preferred_element_type=jnp.float32)
           + jnp.dot(tx1v, Wc[1], preferred_element_type=jnp.float32)
           + jnp.dot(tx2, Wc[2], preferred_element_type=jnp.float32)
           + bc[...])
    mean = jnp.sum(raw, axis=0, keepdims=True) * (1.0 / _N)
    var = jnp.sum(raw * raw, axis=0, keepdims=True) * (1.0 / _N) - mean * mean
    inv = lax.rsqrt(var + _EPS)
    y = (raw - mean) * inv * gamma[...] + beta[...]
    out_o[...] = jnp.maximum(y, 0.0) + ident[...]


def _tc3(S2, dinv, x, hs2, W_cheb, b_cheb, ident, gamma, beta):
    return pl.pallas_call(
        _tc3_body,
        out_shape=jax.ShapeDtypeStruct((_N, _C), jnp.float32),
        compiler_params=pltpu.CompilerParams(
            vmem_limit_bytes=100 * 1024 * 1024),
    )(S2, dinv, x, hs2, W_cheb, b_cheb, ident, gamma, beta)


def kernel(edge_index, x, W_cheb, b_cheb, gamma, beta, W_match, b_match):
    ei = edge_index.astype(jnp.int32)
    pad = _N + (jnp.arange(_EPAD - _E, dtype=jnp.int32) % (_NP - _N))
    row2d = jnp.concatenate([ei[0], pad]).reshape(_NCHUNK, _B)
    col2d = jnp.concatenate([ei[1], pad]).reshape(_NCHUNK, _B)

    ident = _tcid(x, W_match, b_match.reshape(1, _C))
    degp = _deg_kernel()(row2d)
    dinv, hs1 = _tc1(degp[:_DR].reshape(_NP, 1),
                     degp[_DR:].reshape(_NP, 1), x)
    S1 = _prop_kernel()(hs1, col2d, row2d).reshape(_NC, _NP, _C)
    hs2 = _tc2(S1, dinv)
    S2 = _prop_kernel()(hs2, col2d, row2d).reshape(_NC, _NP, _C)
    return _tc3(S2, dinv, x, hs2, W_cheb, b_cheb.reshape(1, _C),
                ident, gamma.reshape(1, _C), beta.reshape(1, _C))


# speedup vs baseline: 24.9363x; 1.0065x over previous
import functools

import jax
import jax.numpy as jnp
from jax import lax
from jax.experimental import pallas as pl
from jax.experimental.pallas import tpu as pltpu
from jax.experimental.pallas import tpu_sc as plsc

_N = 10000
_E = 320000
_C = 128
_EPS = 1e-5
_NP = 10240
_B = 128
_NCHUNK = 2560
_EPAD = _NCHUNK * _B
_NC = 2
_NS = 16
_GT = _NCHUNK // (_NC * _NS)
_GH = _GT // 2
_ZR = _NP // _NS
_DW = 16

_DR = _NP // _B


def _deg_body(row2d, out, accd, rowv, deg1, deg2, idv):
    c = lax.axis_index("c")
    s = lax.axis_index("s")

    def zfill(i, carry):
        deg1[pl.ds(i * 16, 16)] = jnp.zeros((16,), jnp.float32)
        return carry

    lax.fori_loop(0, _NP // 16, zfill, 0)
    for k in range(_DR // 16):
        idv[0, pl.ds(k * 16, 16)] = lax.iota(jnp.int32, 16) + (k * 16)

    def z2(i, carry):
        for j in range(_B // 16):
            deg2[i, pl.ds(j * 16, 16)] = jnp.zeros((16,), jnp.float32)
        return carry

    lax.fori_loop(0, 8, z2, 0)

    @pl.when(s < _DR // 8)
    def _zero_acc():
        pltpu.sync_copy(deg2.at[pl.ds(0, 8)], accd.at[pl.ds(s * 8, 8)])

    base = (c * _NS + s) * _GT
    pltpu.sync_copy(row2d.at[pl.ds(base, _GT)], rowv)
    plsc.subcore_barrier()

    ones16 = jnp.full((16,), 1.0, jnp.float32)

    def hist(g, carry):
        for j in range(_B // 16):
            idx = rowv[g, pl.ds(j * 16, 16)]
            plsc.addupdate_scatter(deg1, [idx], ones16)
        return carry

    lax.fori_loop(0, _GT, hist, 0)

    def tw(g, carry):
        for j in range(_B // 16):
            deg2[g, pl.ds(j * 16, 16)] = deg1[pl.ds(g * _B + j * 16, 16)]
        return carry

    lax.fori_loop(0, _DR, tw, 0)
    pltpu.sync_copy(deg2, accd.at[idv.at[0]], add=True)
    plsc.subcore_barrier()

    @pl.when(s < _DR // 8)
    def _dump():
        pltpu.sync_copy(accd.at[pl.ds(s * 8, 8)],
                        out.at[pl.ds(c * _DR + s * 8, 8)])


@functools.cache
def _deg_kernel():
    mesh = plsc.VectorSubcoreMesh(core_axis_name="c", subcore_axis_name="s")
    return pl.kernel(
        _deg_body,
        out_type=jax.ShapeDtypeStruct((_NC * _DR, _B), jnp.float32),
        mesh=mesh,
        compiler_params=pltpu.CompilerParams(needs_layout_passes=False),
        scratch_types=[
            pltpu.VMEM_SHARED((_DR, _B), jnp.float32),
            pltpu.VMEM((_GT, _B), jnp.int32),
            pltpu.VMEM((_NP,), jnp.float32),
            pltpu.VMEM((_DR, _B), jnp.float32),
            pltpu.VMEM((1, _DR), jnp.int32),
        ],
    )


_ZB = 32


def _prop_body(hs, col2d, row2d, out, acc, colv, rowv, gbuf0, gbuf1, zbuf,
               gsem0, gsem1):
    c = lax.axis_index("c")
    s = lax.axis_index("s")
    base = (c * _NS + s) * _GT

    pltpu.sync_copy(col2d.at[pl.ds(base, _GH)], colv)
    pltpu.sync_copy(row2d.at[pl.ds(base, _GH)], rowv)
    pltpu.async_copy(hs.at[colv.at[0]], gbuf0, gsem0)
    pltpu.async_copy(hs.at[colv.at[1]], gbuf1, gsem1)

    def fill(i, carry):
        for j in range(_C // 16):
            zbuf[i, pl.ds(j * 16, 16)] = jnp.zeros((16,), jnp.float32)
        return carry

    lax.fori_loop(0, _ZB, fill, 0)

    def zero(k, carry):
        pltpu.sync_copy(zbuf, acc.at[pl.ds(s * _ZR + k * _ZB, _ZB)])
        return carry

    lax.fori_loop(0, _ZR // _ZB, zero, 0)
    plsc.subcore_barrier()

    def half(hh, carry):
        hbase = base + hh * _GH

        @pl.when(hh > 0)
        def _stage():
            pltpu.sync_copy(col2d.at[pl.ds(hbase, _GH)], colv)
            pltpu.sync_copy(row2d.at[pl.ds(hbase, _GH)], rowv)
            pltpu.async_copy(hs.at[colv.at[0]], gbuf0, gsem0)
            pltpu.async_copy(hs.at[colv.at[1]], gbuf1, gsem1)

        def chunk2(h, carry2):
            for (buf, sem, off) in ((gbuf0, gsem0, 0), (gbuf1, gsem1, 1)):
                g = h * 2 + off
                pltpu.make_async_copy(hs.at[colv.at[g]], buf, sem).wait()
                pltpu.sync_copy(buf, acc.at[rowv.at[g]], add=True)

                @pl.when(g + 2 < _GH)
                def _refill():
                    pltpu.async_copy(hs.at[colv.at[g + 2]], buf, sem)

            return carry2

        lax.fori_loop(0, _GH // 2, chunk2, 0)
        return carry

    lax.fori_loop(0, _GT // _GH, half, 0)
    plsc.subcore_barrier()
    pltpu.sync_copy(acc.at[pl.ds(s * _ZR, _ZR)],
                    out.at[pl.ds(c * _NP + s * _ZR, _ZR)])


@functools.cache
def _prop_kernel():
    mesh = plsc.VectorSubcoreMesh(core_axis_name="c", subcore_axis_name="s")
    return pl.kernel(
        _prop_body,
        out_type=jax.ShapeDtypeStruct((_NC * _NP, _C), jnp.float32),
        mesh=mesh,
        scratch_types=[
            pltpu.VMEM_SHARED((_NP, _C), jnp.float32),
            pltpu.VMEM((_GH, _B), jnp.int32),
            pltpu.VMEM((_GH, _B), jnp.int32),
            pltpu.VMEM((_B, _C), jnp.float32),
            pltpu.VMEM((_B, _C), jnp.float32),
            pltpu.VMEM((_ZB, _C), jnp.float32),
            pltpu.SemaphoreType.DMA,
            pltpu.SemaphoreType.DMA,
        ],
    )


def _tc1_body(degA, degB, x, dinv_o, hs1_o):
    d = degA[...] + degB[...]
    dinv = jnp.where(d > 0.0, lax.rsqrt(d), 0.0)
    dinv_o[...] = dinv
    hs1_o[pl.ds(0, _N), :] = x[...] * dinv[:_N]


def _tc1(degA, degB, x):
    return pl.pallas_call(
        _tc1_body,
        out_shape=[
            jax.ShapeDtypeStruct((_NP, 1), jnp.float32),
            jax.ShapeDtypeStruct((_NP, _C), jnp.float32),
        ],
    )(degA, degB, x)


def _tc2_body(S, dinv, hs2_o):
    ssum = S[0, pl.ds(0, _N), :] + S[1, pl.ds(0, _N), :]
    d = dinv[pl.ds(0, _N)]
    hs2_o[pl.ds(0, _N), :] = -(d * d * ssum)


def _tc2(S1, dinv):
    return pl.pallas_call(
        _tc2_body,
        out_shape=jax.ShapeDtypeStruct((_NP, _C), jnp.float32),
    )(S1, dinv)


def _tc3_body(S, dinv, x, hs2, Wc, bc, Wm, bm, gamma, beta, out_o):
    ssum = S[0, pl.ds(0, _N), :] + S[1, pl.ds(0, _N), :]
    d = dinv[pl.ds(0, _N)]
    xv = x[...]
    tx1v = hs2[pl.ds(0, _N), :] * jnp.where(d > 0.0, 1.0 / d, 0.0)
    tx2 = -2.0 * (d * ssum) - xv
    raw = (jnp.dot(xv, Wc[0], preferred_element_type=jnp.float32)
           + jnp.dot(tx1v, Wc[1], preferred_element_type=jnp.float32)
           + jnp.dot(tx2, Wc[2], preferred_element_type=jnp.float32)
           + bc[...])
    mean = jnp.sum(raw, axis=0, keepdims=True) * (1.0 / _N)
    var = jnp.sum(raw * raw, axis=0, keepdims=True) * (1.0 / _N) - mean * mean
    inv = lax.rsqrt(var + _EPS)
    y = (raw - mean) * inv * gamma[...] + beta[...]
    ident = jnp.dot(xv, Wm[...], preferred_element_type=jnp.float32) + bm[...]
    out_o[...] = jnp.maximum(y, 0.0) + ident


def _tc3(S2, dinv, x, hs2, W_cheb, b_cheb, W_match, b_match, gamma, beta):
    return pl.pallas_call(
        _tc3_body,
        out_shape=jax.ShapeDtypeStruct((_N, _C), jnp.float32),
        compiler_params=pltpu.CompilerParams(
            vmem_limit_bytes=100 * 1024 * 1024),
    )(S2, dinv, x, hs2, W_cheb, b_cheb, W_match, b_match, gamma, beta)


def kernel(edge_index, x, W_cheb, b_cheb, gamma, beta, W_match, b_match):
    ei = edge_index.astype(jnp.int32)
    pad = _N + (jnp.arange(_EPAD - _E, dtype=jnp.int32) % (_NP - _N))
    row2d = jnp.concatenate([ei[0], pad]).reshape(_NCHUNK, _B)
    col2d = jnp.concatenate([ei[1], pad]).reshape(_NCHUNK, _B)

    degp = _deg_kernel()(row2d)
    dinv, hs1 = _tc1(degp[:_DR].reshape(_NP, 1),
                     degp[_DR:].reshape(_NP, 1), x)
    S1 = _prop_kernel()(hs1, col2d, row2d).reshape(_NC, _NP, _C)
    hs2 = _tc2(S1, dinv)
    S2 = _prop_kernel()(hs2, col2d, row2d).reshape(_NC, _NP, _C)
    return _tc3(S2, dinv, x, hs2, W_cheb, b_cheb.reshape(1, _C),
                W_match, b_match.reshape(1, _C),
                gamma.reshape(1, _C), beta.reshape(1, _C))
